# Initial kernel scaffold; baseline (speedup 1.0000x reference)
import jax
import jax.numpy as jnp
from jax.experimental import pallas as pl


def kernel(inputs, embedding):
    raise NotImplementedError("write your pallas kernel here")


# speedup vs baseline: 1.1811x; 1.1811x over previous
import functools

import jax
import jax.numpy as jnp
from jax import lax
from jax.experimental import pallas as pl
from jax.experimental.pallas import tpu as pltpu
from jax.experimental.pallas import tpu_sc as plsc

_NE = 1024
_D = 128
_N = 16 * 576
_BR = 512
_SCALE = 1.25 / (_N * _D)

_NW = 32
_BPW = _N // _NW


def _tc_body(x_ref, e_ref, idx_ref, loss_ref, acc_ref):
    i = pl.program_id(0)
    xb = x_ref[...]
    e = e_ref[...]
    a = jnp.sum(xb * xb, axis=1, keepdims=True)
    b = jnp.sum(e * e, axis=1)[None, :]
    m = lax.dot_general(xb, e, (((1,), (1,)), ((), ())),
                        preferred_element_type=jnp.float32)
    d = (a + b) - 2.0 * m
    rowmin = jnp.min(d, axis=1, keepdims=True)
    col = lax.broadcasted_iota(jnp.int32, d.shape, 1)
    idx = jnp.min(jnp.where(d == rowmin, col, jnp.int32(_NE)), axis=1)
    idx_ref[...] = idx

    @pl.when(i == 0)
    def _():
        acc_ref[0] = 0.0

    acc_ref[0] += jnp.sum(rowmin)

    @pl.when(i == pl.num_programs(0) - 1)
    def _():
        loss_ref[0] = acc_ref[0] * _SCALE


def _tc_distance_argmin(flat, emb):
    return pl.pallas_call(
        _tc_body,
        grid=(_N // _BR,),
        in_specs=[
            pl.BlockSpec((_BR, _D), lambda i: (i, 0)),
            pl.BlockSpec((_NE, _D), lambda i: (0, 0)),
        ],
        out_specs=[
            pl.BlockSpec((_BR,), lambda i: (i,)),
            pl.BlockSpec(memory_space=pltpu.SMEM),
        ],
        out_shape=[
            jax.ShapeDtypeStruct((_N,), jnp.int32),
            jax.ShapeDtypeStruct((1,), jnp.float32),
        ],
        scratch_shapes=[pltpu.SMEM((1,), jnp.float32)],
    )(flat, emb)


def _sc_gather(emb, idx):
    mesh = plsc.VectorSubcoreMesh(core_axis_name="c", subcore_axis_name="s")

    @functools.partial(
        pl.kernel,
        mesh=mesh,
        out_type=jax.ShapeDtypeStruct((_N, _D), jnp.float32),
        scratch_types=[
            pltpu.VMEM((_BPW,), jnp.int32),
            pltpu.VMEM((_BPW, _D), jnp.float32),
            pltpu.SemaphoreType.DMA,
        ],
    )
    def k(table_hbm, idx_hbm, out_hbm, idx_v, rows_v, sem):
        wid = lax.axis_index("s") * 2 + lax.axis_index("c")
        base = wid * _BPW
        pltpu.sync_copy(idx_hbm.at[pl.ds(base, _BPW)], idx_v)
        pltpu.async_copy(table_hbm.at[idx_v], rows_v, sem).wait()
        pltpu.sync_copy(rows_v, out_hbm.at[pl.ds(base, _BPW)])

    return k(emb, idx)


def kernel(inputs, embedding):
    flat = inputs.reshape(_N, _D)
    idx, loss = _tc_distance_argmin(flat, embedding)
    qflat = _sc_gather(embedding, idx)
    return (loss[0], qflat.reshape(inputs.shape), idx)


# speedup vs baseline: 1.2762x; 1.0806x over previous
import functools

import jax
import jax.numpy as jnp
from jax import lax
from jax.experimental import pallas as pl
from jax.experimental.pallas import tpu as pltpu
from jax.experimental.pallas import tpu_sc as plsc

_NE = 1024
_D = 128
_N = 16 * 576
_BR = 512
_LC = 128
_NC = _NE // _LC
_SCALE = 1.25 / (_N * _D)

_NW = 32
_BPW = _N // _NW


def _tc_body(x_ref, e2_ref, b_ref, idx_ref, loss_ref, acc_ref):
    i = pl.program_id(0)
    xb = x_ref[...]
    a = jnp.sum(xb * xb, axis=1, keepdims=True)
    b = b_ref[...][None, :]
    m2 = lax.dot_general(xb, e2_ref[...], (((1,), (1,)), ((), ())),
                         preferred_element_type=jnp.float32)
    d = (a + b) - m2
    v = d[:, 0:_LC]
    for k in range(1, _NC):
        v = jnp.minimum(v, d[:, k * _LC:(k + 1) * _LC])
    rowmin = jnp.min(v, axis=1, keepdims=True)
    col = lax.broadcasted_iota(jnp.int32, (_BR, _LC), 1).astype(jnp.float32)
    ii = jnp.where(d[:, 0:_LC] == rowmin, col, jnp.float32(_NE))
    for k in range(1, _NC):
        cand = jnp.where(d[:, k * _LC:(k + 1) * _LC] == rowmin,
                         col + jnp.float32(k * _LC), jnp.float32(_NE))
        ii = jnp.minimum(ii, cand)
    idx_ref[...] = jnp.min(ii, axis=1, keepdims=True).astype(jnp.int32)

    @pl.when(i == 0)
    def _():
        acc_ref[0] = 0.0

    acc_ref[0] += jnp.sum(rowmin)

    @pl.when(i == pl.num_programs(0) - 1)
    def _():
        loss_ref[0] = acc_ref[0] * _SCALE


def _tc_distance_argmin(flat, e2, b):
    return pl.pallas_call(
        _tc_body,
        grid=(_N // _BR,),
        in_specs=[
            pl.BlockSpec((_BR, _D), lambda i: (i, 0)),
            pl.BlockSpec((_NE, _D), lambda i: (0, 0)),
            pl.BlockSpec((_NE,), lambda i: (0,)),
        ],
        out_specs=[
            pl.BlockSpec((_BR, 1), lambda i: (i, 0)),
            pl.BlockSpec(memory_space=pltpu.SMEM),
        ],
        out_shape=[
            jax.ShapeDtypeStruct((_N, 1), jnp.int32),
            jax.ShapeDtypeStruct((1,), jnp.float32),
        ],
        scratch_shapes=[pltpu.SMEM((1,), jnp.float32)],
    )(flat, e2, b)


def _sc_gather(emb, idx):
    mesh = plsc.VectorSubcoreMesh(core_axis_name="c", subcore_axis_name="s")

    @functools.partial(
        pl.kernel,
        mesh=mesh,
        out_type=jax.ShapeDtypeStruct((_N, _D), jnp.float32),
        scratch_types=[
            pltpu.VMEM((_BPW,), jnp.int32),
            pltpu.VMEM((_BPW, _D), jnp.float32),
            pltpu.SemaphoreType.DMA,
        ],
    )
    def k(table_hbm, idx_hbm, out_hbm, idx_v, rows_v, sem):
        wid = lax.axis_index("s") * 2 + lax.axis_index("c")
        base = wid * _BPW
        pltpu.sync_copy(idx_hbm.at[pl.ds(base, _BPW)], idx_v)
        pltpu.async_copy(table_hbm.at[idx_v], rows_v, sem).wait()
        pltpu.sync_copy(rows_v, out_hbm.at[pl.ds(base, _BPW)])

    return k(emb, idx)


def kernel(inputs, embedding):
    flat = inputs.reshape(_N, _D)
    e2 = embedding + embedding
    b = jnp.sum(embedding ** 2, axis=1)
    idx2, loss = _tc_distance_argmin(flat, e2, b)
    idx = idx2.reshape(_N)
    qflat = _sc_gather(embedding, idx)
    return (loss[0], qflat.reshape(inputs.shape), idx)


# speedup vs baseline: 1.3273x; 1.0400x over previous
import functools

import jax
import jax.numpy as jnp
from jax import lax
from jax.experimental import pallas as pl
from jax.experimental.pallas import tpu as pltpu
from jax.experimental.pallas import tpu_sc as plsc

_NE = 1024
_D = 128
_N = 16 * 576
_BR = 512
_LC = 128
_NC = _NE // _LC
_SCALE = 1.25 / (_N * _D)

_NW = 32
_BPW = _N // _NW


def _tc_body(x_ref, e_ref, idx_ref, loss_ref, e2_ref, b_ref, acc_ref):
    i = pl.program_id(0)

    @pl.when(i == 0)
    def _():
        e = e_ref[...]
        e2_ref[...] = e + e
        b_ref[...] = jnp.sum(e * e, axis=1)[None, :]
        acc_ref[0] = 0.0

    xb = x_ref[...]
    a = jnp.sum(xb * xb, axis=1, keepdims=True)
    b = b_ref[...]
    m2 = lax.dot_general(xb, e2_ref[...], (((1,), (1,)), ((), ())),
                         preferred_element_type=jnp.float32)
    d = (a + b) - m2
    v = d[:, 0:_LC]
    for k in range(1, _NC):
        v = jnp.minimum(v, d[:, k * _LC:(k + 1) * _LC])
    rowmin = jnp.min(v, axis=1, keepdims=True)
    col = lax.broadcasted_iota(jnp.int32, (_BR, _LC), 1).astype(jnp.float32)
    ii = jnp.where(d[:, 0:_LC] == rowmin, col, jnp.float32(_NE))
    for k in range(1, _NC):
        cand = jnp.where(d[:, k * _LC:(k + 1) * _LC] == rowmin,
                         col + jnp.float32(k * _LC), jnp.float32(_NE))
        ii = jnp.minimum(ii, cand)
    idx_ref[...] = jnp.min(ii, axis=1, keepdims=True).astype(jnp.int32)

    acc_ref[0] += jnp.sum(rowmin)

    @pl.when(i == pl.num_programs(0) - 1)
    def _():
        loss_ref[0] = acc_ref[0] * _SCALE


def _tc_distance_argmin(flat, emb):
    return pl.pallas_call(
        _tc_body,
        grid=(_N // _BR,),
        in_specs=[
            pl.BlockSpec((_BR, _D), lambda i: (i, 0)),
            pl.BlockSpec((_NE, _D), lambda i: (0, 0)),
        ],
        out_specs=[
            pl.BlockSpec((_BR, 1), lambda i: (i, 0)),
            pl.BlockSpec(memory_space=pltpu.SMEM),
        ],
        out_shape=[
            jax.ShapeDtypeStruct((_N, 1), jnp.int32),
            jax.ShapeDtypeStruct((1,), jnp.float32),
        ],
        scratch_shapes=[
            pltpu.VMEM((_NE, _D), jnp.float32),
            pltpu.VMEM((1, _NE), jnp.float32),
            pltpu.SMEM((1,), jnp.float32),
        ],
    )(flat, emb)


def _sc_gather(emb, idx):
    mesh = plsc.VectorSubcoreMesh(core_axis_name="c", subcore_axis_name="s")

    @functools.partial(
        pl.kernel,
        mesh=mesh,
        out_type=jax.ShapeDtypeStruct((_N, _D), jnp.float32),
        scratch_types=[
            pltpu.VMEM((_BPW,), jnp.int32),
            pltpu.VMEM((_BPW, _D), jnp.float32),
            pltpu.SemaphoreType.DMA,
        ],
    )
    def k(table_hbm, idx_hbm, out_hbm, idx_v, rows_v, sem):
        wid = lax.axis_index("s") * 2 + lax.axis_index("c")
        base = wid * _BPW
        pltpu.sync_copy(idx_hbm.at[pl.ds(base, _BPW)], idx_v)
        pltpu.async_copy(table_hbm.at[idx_v], rows_v, sem).wait()
        pltpu.sync_copy(rows_v, out_hbm.at[pl.ds(base, _BPW)])

    return k(emb, idx)


def kernel(inputs, embedding):
    flat = inputs.reshape(_N, _D)
    idx2, loss = _tc_distance_argmin(flat, embedding)
    idx = idx2.reshape(_N)
    qflat = _sc_gather(embedding, idx)
    return (loss[0], qflat.reshape(inputs.shape), idx)


# speedup vs baseline: 1.5693x; 1.1823x over previous
import functools

import jax
import jax.numpy as jnp
from jax import lax
from jax.experimental import pallas as pl
from jax.experimental.pallas import tpu as pltpu
from jax.experimental.pallas import tpu_sc as plsc

_NE = 1024
_D = 128
_N = 16 * 576
_BR = 1024
_LC = 128
_NC = _NE // _LC
_SCALE = 1.25 / (_N * _D)

_NW = 32
_BPW = _N // _NW


def _tc_body(x_ref, e_ref, idx_ref, loss_ref, e2_ref, b_ref, acc_ref):
    i = pl.program_id(0)

    @pl.when(i == 0)
    def _():
        e = e_ref[...]
        e2_ref[...] = e + e
        b_ref[...] = jnp.sum(e * e, axis=1)[None, :]
        acc_ref[0] = 0.0

    xb = x_ref[...]
    a = jnp.sum(xb * xb, axis=1, keepdims=True)
    b = b_ref[...]
    m2 = lax.dot_general(xb, e2_ref[...], (((1,), (1,)), ((), ())),
                         preferred_element_type=jnp.float32)
    d = (a + b) - m2
    v = d[:, 0:_LC]
    for k in range(1, _NC):
        v = jnp.minimum(v, d[:, k * _LC:(k + 1) * _LC])
    rowmin = jnp.min(v, axis=1, keepdims=True)
    col = lax.broadcasted_iota(jnp.int32, (_BR, _LC), 1).astype(jnp.float32)
    ii = jnp.where(d[:, 0:_LC] == rowmin, col, jnp.float32(_NE))
    for k in range(1, _NC):
        cand = jnp.where(d[:, k * _LC:(k + 1) * _LC] == rowmin,
                         col + jnp.float32(k * _LC), jnp.float32(_NE))
        ii = jnp.minimum(ii, cand)
    idx_ref[...] = (jnp.min(ii, axis=1).astype(jnp.int32)
                    .reshape(_BR // _LC, _LC))

    acc_ref[0] += jnp.sum(rowmin)

    @pl.when(i == pl.num_programs(0) - 1)
    def _():
        loss_ref[0] = acc_ref[0] * _SCALE


def _tc_distance_argmin(flat, emb):
    return pl.pallas_call(
        _tc_body,
        grid=(_N // _BR,),
        in_specs=[
            pl.BlockSpec((_BR, _D), lambda i: (i, 0)),
            pl.BlockSpec((_NE, _D), lambda i: (0, 0)),
        ],
        out_specs=[
            pl.BlockSpec((_BR // _LC, _LC), lambda i: (i, 0)),
            pl.BlockSpec(memory_space=pltpu.SMEM),
        ],
        out_shape=[
            jax.ShapeDtypeStruct((_N // _LC, _LC), jnp.int32),
            jax.ShapeDtypeStruct((1,), jnp.float32),
        ],
        scratch_shapes=[
            pltpu.VMEM((_NE, _D), jnp.float32),
            pltpu.VMEM((1, _NE), jnp.float32),
            pltpu.SMEM((1,), jnp.float32),
        ],
    )(flat, emb)


def _sc_gather(emb, idx):
    mesh = plsc.VectorSubcoreMesh(core_axis_name="c", subcore_axis_name="s")

    @functools.partial(
        pl.kernel,
        mesh=mesh,
        out_type=jax.ShapeDtypeStruct((_N, _D), jnp.float32),
        scratch_types=[
            pltpu.VMEM((_BPW,), jnp.int32),
            pltpu.VMEM((_BPW, _D), jnp.float32),
            pltpu.SemaphoreType.DMA,
        ],
    )
    def k(table_hbm, idx_hbm, out_hbm, idx_v, rows_v, sem):
        wid = lax.axis_index("s") * 2 + lax.axis_index("c")
        base = wid * _BPW
        pltpu.sync_copy(idx_hbm.at[pl.ds(base, _BPW)], idx_v)
        pltpu.async_copy(table_hbm.at[idx_v], rows_v, sem).wait()
        pltpu.sync_copy(rows_v, out_hbm.at[pl.ds(base, _BPW)])

    return k(emb, idx)


def kernel(inputs, embedding):
    flat = inputs.reshape(_N, _D)
    idx2, loss = _tc_distance_argmin(flat, embedding)
    idx = idx2.reshape(_N)
    qflat = _sc_gather(embedding, idx)
    return (loss[0], qflat.reshape(inputs.shape), idx)
